# baseline (speedup 1.0000x reference)
import functools

import numpy as np
import jax
import jax.numpy as jnp
from jax.experimental import pallas as pl
from jax.experimental.pallas import tpu as pltpu
from jax.experimental.pallas import tpu_sc as plsc

_K = 64
_ROWS = 128
_COLS = 32768
_BLK_ROWS = 8
_TILE = 2048
_NT = _COLS // _TILE
_NW = 32
_RPW = _ROWS // _NW
_LANE_BUF = 2048


def _keys_body(scores_ref, tau_ref, g_ref, out_ref):
    tau = tau_ref[0, 0]
    for tj in range(_NT):
        sl = pl.ds(tj * _TILE, _TILE)
        sample = scores_ref[:, sl] / tau + g_ref[:, sl]
        ub = jax.lax.bitcast_convert_type(sample, jnp.uint32)
        skey = ub ^ ((ub >> jnp.uint32(31)) * jnp.uint32(0x7FFFFFFF))
        out_ref[:, sl] = jax.lax.bitcast_convert_type(skey, jnp.int32)


def _sc_body(keys_hbm, out_hbm, row_v, hist1, hist2, candk, candc):
    cid = jax.lax.axis_index("c")
    sid = jax.lax.axis_index("s")
    wid = sid * 2 + cid
    lane = jax.lax.iota(jnp.int32, 16)
    ones = jnp.full((16,), 1, jnp.int32)
    zeros16 = jnp.zeros((16,), jnp.int32)

    def row_step(ri, _carry):
        r = wid * _RPW + ri
        pltpu.sync_copy(keys_hbm.at[r], row_v)

        @plsc.parallel_loop(0, 512, unroll=8)
        def _(j):
            hist1[pl.ds(j * 16, 16)] = zeros16

        @plsc.parallel_loop(0, 2048, unroll=16)
        def _(j):
            k = row_v[pl.ds(j * 16, 16)]
            d = jax.lax.shift_right_arithmetic(k, 23) + 256
            plsc.addupdate_scatter(hist1, [d * 16 + lane], ones)

        def s1_cond(c):
            _d, acc = c
            return acc < _K
        def s1_body(c):
            d, acc = c
            cnt = jnp.sum(hist1[pl.ds(d * 16, 16)])
            return (d - 1, acc + cnt)
        d_end, acc1 = jax.lax.while_loop(
            s1_cond, s1_body, (jnp.int32(511), jnp.int32(0)))
        b1 = d_end + 1
        cnt_b1 = jnp.sum(hist1[pl.ds(b1 * 16, 16)])
        needed1 = _K - (acc1 - cnt_b1)

        @plsc.parallel_loop(0, 64, unroll=8)
        def _(j):
            hist2[pl.ds(j * 16, 16)] = zeros16

        @plsc.parallel_loop(0, 2048, unroll=16)
        def _(j):
            k = row_v[pl.ds(j * 16, 16)]
            d = jax.lax.shift_right_arithmetic(k, 23) + 256
            e = jax.lax.shift_right_logical(k, 17) & 63
            plsc.addupdate_scatter(hist2, [e * 16 + lane], ones, mask=d == b1)

        def s2_cond(c):
            _e, acc = c
            return acc < needed1
        def s2_body(c):
            e, acc = c
            cnt = jnp.sum(hist2[pl.ds(e * 16, 16)])
            return (e - 1, acc + cnt)
        e_end, acc2 = jax.lax.while_loop(
            s2_cond, s2_body, (jnp.int32(63), jnp.int32(0)))
        b2 = e_end + 1
        cnt_b2 = jnp.sum(hist2[pl.ds(b2 * 16, 16)])
        needed2 = needed1 - (acc2 - cnt_b2)
        p15 = b1 * 64 + b2

        def cp(j, pos):
            k = row_v[pl.ds(j * 16, 16)]
            m = (jax.lax.shift_right_arithmetic(k, 17) + 16384) == p15
            idx = lane * _LANE_BUF + pos
            plsc.store_scatter(candk, [idx], k, mask=m)
            plsc.store_scatter(candc, [idx], j * 16 + lane, mask=m)
            return pos + m.astype(jnp.int32)
        lens = plsc.parallel_loop(0, 2048, unroll=16, carry=zeros16)(cp)
        maxlen = jnp.max(lens)

        def cnt_lo_ge(th):
            def ib(j, a):
                idx = lane * _LANE_BUF + j
                ck = plsc.load_gather(candk, [idx])
                ok = ((ck & 0x1FFFF) >= th) & (j < lens)
                return a + ok.astype(jnp.int32)
            return jnp.sum(jax.lax.fori_loop(0, maxlen, ib, zeros16))

        def tb(i, tl):
            cand = tl | (jnp.int32(1) << (16 - i))
            return jnp.where(cnt_lo_ge(cand) >= needed2, cand, tl)
        t_lo = jax.lax.fori_loop(0, 17, tb, jnp.int32(0))
        n_ties = needed2 - cnt_lo_ge(t_lo + 1)
        t_full = ((p15 - 16384) << 17) | t_lo

        def cnt_col_lt(cv):
            def ib(j, a):
                idx = lane * _LANE_BUF + j
                ck = plsc.load_gather(candk, [idx])
                cc = plsc.load_gather(candc, [idx])
                ok = (ck == t_full) & (cc < cv) & (j < lens)
                return a + ok.astype(jnp.int32)
            return jnp.sum(jax.lax.fori_loop(0, maxlen, ib, zeros16))

        def pb(i, pv):
            cand = pv | (jnp.int32(1) << (14 - i))
            return jnp.where(cnt_col_lt(cand) < n_ties, cand, pv)
        p = jax.lax.fori_loop(0, 15, pb, jnp.int32(0))

        @plsc.parallel_loop(0, 2048, unroll=16)
        def _(j):
            k = row_v[pl.ds(j * 16, 16)]
            col = j * 16 + lane
            sel = (k > t_full) | ((k == t_full) & (col <= p))
            candk[pl.ds(j * 16, 16)] = jnp.where(sel, 0x3F800000, 0)
        pltpu.sync_copy(candk, out_hbm.at[r])
        return _carry

    jax.lax.fori_loop(0, _RPW, row_step, 0)


_sc_select = functools.partial(
    pl.kernel,
    out_type=jax.ShapeDtypeStruct((_ROWS, _COLS), jnp.int32),
    mesh=plsc.VectorSubcoreMesh(core_axis_name="c", subcore_axis_name="s"),
    scratch_types=[
        pltpu.VMEM((_COLS,), jnp.int32),
        pltpu.VMEM((512 * 16,), jnp.int32),
        pltpu.VMEM((64 * 16,), jnp.int32),
        pltpu.VMEM((16 * _LANE_BUF,), jnp.int32),
        pltpu.VMEM((16 * _LANE_BUF,), jnp.int32),
    ],
    compiler_params=pltpu.CompilerParams(needs_layout_passes=False),
)(_sc_body)


_G_CONST = jax.random.gumbel(jax.random.key(42), (_ROWS, _COLS), jnp.float32)


def kernel(scores, tau):
    keys = pl.pallas_call(
        _keys_body,
        grid=(_ROWS // _BLK_ROWS,),
        in_specs=[
            pl.BlockSpec((_BLK_ROWS, _COLS), lambda i: (i, 0)),
            pl.BlockSpec(memory_space=pltpu.SMEM),
            pl.BlockSpec((_BLK_ROWS, _COLS), lambda i: (i, 0)),
        ],
        out_specs=pl.BlockSpec((_BLK_ROWS, _COLS), lambda i: (i, 0)),
        out_shape=jax.ShapeDtypeStruct((_ROWS, _COLS), jnp.int32),
    )(scores, tau.reshape(1, 1), _G_CONST)
    maskbits = _sc_select(keys)
    return jax.lax.bitcast_convert_type(maskbits, jnp.float32)


# speedup vs baseline: 1.5384x; 1.5384x over previous
import functools

import numpy as np
import jax
import jax.numpy as jnp
from jax.experimental import pallas as pl
from jax.experimental.pallas import tpu as pltpu
from jax.experimental.pallas import tpu_sc as plsc

_K = 64
_ROWS = 128
_COLS = 32768
_BLK_ROWS = 8
_TILE = 2048
_NT = _COLS // _TILE
_NW = 32
_RPW = _ROWS // _NW
_LANE_BUF = 2048


def _keys_body(scores_ref, tau_ref, g_ref, out_ref):
    tau = tau_ref[0, 0]
    for tj in range(_NT):
        sl = pl.ds(tj * _TILE, _TILE)
        sample = scores_ref[:, sl] / tau + g_ref[:, sl]
        ub = jax.lax.bitcast_convert_type(sample, jnp.uint32)
        skey = ub ^ ((ub >> jnp.uint32(31)) * jnp.uint32(0x7FFFFFFF))
        out_ref[:, sl] = jax.lax.bitcast_convert_type(skey, jnp.int32)


def _sc_body(keys_hbm, out_hbm, row_a, row_b, hist1, hist2, candp, sem_in, sem_out):
    cid = jax.lax.axis_index("c")
    sid = jax.lax.axis_index("s")
    wid = sid * 2 + cid
    lane = jax.lax.iota(jnp.int32, 16)
    ones = jnp.full((16,), 1, jnp.int32)
    zeros16 = jnp.zeros((16,), jnp.int32)
    bufs = (row_a, row_b)

    pltpu.sync_copy(keys_hbm.at[wid * _RPW], row_a)
    out_dma = None
    for rr in range(_RPW):
        row_v = bufs[rr % 2]
        if rr + 1 < _RPW:
            in_dma = pltpu.async_copy(
                keys_hbm.at[wid * _RPW + rr + 1], bufs[(rr + 1) % 2], sem_in)

        @plsc.parallel_loop(0, 512, unroll=8)
        def _(j):
            hist1[pl.ds(j * 16, 16)] = zeros16

        @plsc.parallel_loop(0, 2048, unroll=8)
        def _(j):
            k = row_v[pl.ds(j * 16, 16)]
            d = jax.lax.shift_right_arithmetic(k, 23) + 256
            plsc.addupdate_scatter(hist1, [d * 16 + lane], ones)

        def s1_cond(c):
            _d, acc = c
            return acc < _K
        def s1_body(c):
            d, acc = c
            cnt = jnp.sum(hist1[pl.ds(d * 16, 16)])
            return (d - 1, acc + cnt)
        d_end, acc1 = jax.lax.while_loop(
            s1_cond, s1_body, (jnp.int32(511), jnp.int32(0)))
        b1 = d_end + 1
        cnt_b1 = jnp.sum(hist1[pl.ds(b1 * 16, 16)])
        needed1 = _K - (acc1 - cnt_b1)

        @plsc.parallel_loop(0, 64, unroll=8)
        def _(j):
            hist2[pl.ds(j * 16, 16)] = zeros16

        @plsc.parallel_loop(0, 2048, unroll=8)
        def _(j):
            k = row_v[pl.ds(j * 16, 16)]
            d = jax.lax.shift_right_arithmetic(k, 23) + 256
            e = jax.lax.shift_right_logical(k, 17) & 63
            plsc.addupdate_scatter(hist2, [e * 16 + lane], ones, mask=d == b1)

        def s2_cond(c):
            _e, acc = c
            return acc < needed1
        def s2_body(c):
            e, acc = c
            cnt = jnp.sum(hist2[pl.ds(e * 16, 16)])
            return (e - 1, acc + cnt)
        e_end, acc2 = jax.lax.while_loop(
            s2_cond, s2_body, (jnp.int32(63), jnp.int32(0)))
        b2 = e_end + 1
        cnt_b2 = jnp.sum(hist2[pl.ds(b2 * 16, 16)])
        needed2 = needed1 - (acc2 - cnt_b2)
        p15 = b1 * 64 + b2

        if out_dma is not None:
            out_dma.wait()

        def cp(j, pos):
            k = row_v[pl.ds(j * 16, 16)]
            m = (jax.lax.shift_right_arithmetic(k, 17) + 16384) == p15
            pv = ((k & 0x1FFFF) << 15) | (32767 - (j * 16 + lane))
            plsc.store_scatter(candp, [lane * _LANE_BUF + pos], pv, mask=m)
            return pos + m.astype(jnp.int32)
        lens = plsc.parallel_loop(0, 2048, unroll=8, carry=zeros16)(cp)
        maxlen = jnp.max(lens)

        def cnt_hi_ge(th):
            def ib(j, a):
                cv = plsc.load_gather(candp, [lane * _LANE_BUF + j])
                hi = jax.lax.shift_right_logical(cv, 15)
                ok = (hi >= th) & (j < lens)
                return a + ok.astype(jnp.int32)
            return jnp.sum(jax.lax.fori_loop(0, maxlen, ib, zeros16))

        def tb(i, tl):
            cand = tl | (jnp.int32(1) << (16 - i))
            return jnp.where(cnt_hi_ge(cand) >= needed2, cand, tl)
        t_lo = jax.lax.fori_loop(0, 17, tb, jnp.int32(0))
        n_ties = needed2 - cnt_hi_ge(t_lo + 1)
        t_full = ((p15 - 16384) << 17) | t_lo

        def cnt_fc_ge(fc):
            def ib(j, a):
                cv = plsc.load_gather(candp, [lane * _LANE_BUF + j])
                ok = (jax.lax.shift_right_logical(cv, 15) == t_lo) \
                    & ((cv & 32767) >= fc) & (j < lens)
                return a + ok.astype(jnp.int32)
            return jnp.sum(jax.lax.fori_loop(0, maxlen, ib, zeros16))

        def pb(i, fc):
            cand = fc | (jnp.int32(1) << (14 - i))
            return jnp.where(cnt_fc_ge(cand) >= n_ties, cand, fc)
        fcut = jax.lax.fori_loop(0, 15, pb, jnp.int32(0))
        p = 32767 - fcut

        @plsc.parallel_loop(0, 2048, unroll=8)
        def _(j):
            k = row_v[pl.ds(j * 16, 16)]
            col = j * 16 + lane
            sel = (k > t_full) | ((k == t_full) & (col <= p))
            candp[pl.ds(j * 16, 16)] = jnp.where(sel, 0x3F800000, 0)
        out_dma = pltpu.async_copy(candp, out_hbm.at[wid * _RPW + rr], sem_out)
        if rr + 1 < _RPW:
            in_dma.wait()
    out_dma.wait()


_sc_select = functools.partial(
    pl.kernel,
    out_type=jax.ShapeDtypeStruct((_ROWS, _COLS), jnp.int32),
    mesh=plsc.VectorSubcoreMesh(core_axis_name="c", subcore_axis_name="s"),
    scratch_types=[
        pltpu.VMEM((_COLS,), jnp.int32),
        pltpu.VMEM((_COLS,), jnp.int32),
        pltpu.VMEM((512 * 16,), jnp.int32),
        pltpu.VMEM((64 * 16,), jnp.int32),
        pltpu.VMEM((16 * _LANE_BUF,), jnp.int32),
        pltpu.SemaphoreType.DMA,
        pltpu.SemaphoreType.DMA,
    ],
    compiler_params=pltpu.CompilerParams(needs_layout_passes=False),
)(_sc_body)


_G_CONST = jax.random.gumbel(jax.random.key(42), (_ROWS, _COLS), jnp.float32)


def kernel(scores, tau):
    keys = pl.pallas_call(
        _keys_body,
        grid=(_ROWS // _BLK_ROWS,),
        in_specs=[
            pl.BlockSpec((_BLK_ROWS, _COLS), lambda i: (i, 0)),
            pl.BlockSpec(memory_space=pltpu.SMEM),
            pl.BlockSpec((_BLK_ROWS, _COLS), lambda i: (i, 0)),
        ],
        out_specs=pl.BlockSpec((_BLK_ROWS, _COLS), lambda i: (i, 0)),
        out_shape=jax.ShapeDtypeStruct((_ROWS, _COLS), jnp.int32),
    )(scores, tau.reshape(1, 1), _G_CONST)
    maskbits = _sc_select(keys)
    return jax.lax.bitcast_convert_type(maskbits, jnp.float32)
